# baseline (speedup 1.0000x reference)
import functools

import jax
import jax.numpy as jnp
from jax import lax
from jax.experimental import pallas as pl
from jax.experimental.pallas import tpu as pltpu
from jax.experimental.pallas import tpu_sc as plsc

_NC = 2
_NS = 16
_LANES = 16
_KCH = 128
_CPAD = 24
_BBLK = 16


def _combine_sc(xs, gbc):
    ps, bb, cc, ll, dd = xs.shape
    nq = dd // _KCH
    bpw = bb // (_NC * _NS)
    nk = dd // _LANES
    mesh = plsc.VectorSubcoreMesh(core_axis_name="c", subcore_axis_name="s")

    @functools.partial(
        pl.kernel,
        out_type=jax.ShapeDtypeStruct((nq, bb * _CPAD, _KCH), jnp.float32),
        mesh=mesh,
        scratch_types=[
            pltpu.VMEM((2, ps, cc, dd), jnp.float32),
            pltpu.VMEM((bpw, ps, _LANES), jnp.float32),
            pltpu.VMEM((2, nq, _CPAD, _KCH), jnp.float32),
            pltpu.SemaphoreType.DMA((2,)),
            pltpu.SemaphoreType.DMA((2,)),
        ],
        compiler_params=pltpu.CompilerParams(use_tc_tiling_on_sc=True),
    )
    def k(xs_hbm, gbc_hbm, comb_hbm, xbuf, gbuf, obuf, insem, outsem):
        wid = lax.axis_index("s") * _NC + lax.axis_index("c")
        b0 = wid * bpw

        def in_copy(slot, j, i):
            return pltpu.make_async_copy(
                xs_hbm.at[i, b0 + j, :, ll - 1, :], xbuf.at[slot, i],
                insem.at[slot])

        def out_copy(slot, j, q):
            return pltpu.make_async_copy(
                obuf.at[slot, q],
                comb_hbm.at[q, pl.ds((b0 + j) * _CPAD, _CPAD), :],
                outsem.at[slot])

        pltpu.sync_copy(gbc_hbm.at[pl.ds(b0, bpw)], gbuf)
        for i in range(ps):
            in_copy(0, 0, i).start()
        for i in range(ps):
            in_copy(1, 1, i).start()

        for j in range(bpw):
            s = j % 2
            for i in range(ps):
                in_copy(s, j, i).wait()
            if j >= 2:
                for q in range(nq):
                    out_copy(s, j - 2, q).wait()
            g = [jnp.maximum(gbuf[j, i], 0.0) for i in range(ps)]

            def do_c(c, carry, s=s, g=g):
                for kk in range(nk):
                    sl = pl.ds(kk * _LANES, _LANES)
                    acc = xbuf[s, 0, c, sl] * g[0] + 1e-9
                    for i in range(1, ps):
                        acc = acc + xbuf[s, i, c, sl] * g[i]
                    obuf[s, kk // (_KCH // _LANES), c,
                         pl.ds((kk % (_KCH // _LANES)) * _LANES, _LANES)] = acc
                return carry

            lax.fori_loop(0, cc, do_c, 0)
            for q in range(nq):
                out_copy(s, j, q).start()
            if j + 2 < bpw:
                for i in range(ps):
                    in_copy(s, j + 2, i).start()

        for q in range(nq):
            out_copy((bpw - 2) % 2, bpw - 2, q).wait()
        for q in range(nq):
            out_copy((bpw - 1) % 2, bpw - 1, q).wait()

    return k(xs, gbc)


def _head_kernel(nq, cc, x_ref, wt_ref, b_ref, o_ref):
    rows = x_ref.shape[1]
    res = jax.lax.dot_general(
        x_ref[0], wt_ref[0:_KCH, :], (((1,), (0,)), ((), ())),
        preferred_element_type=jnp.float32)
    for q in range(1, nq):
        res = res + jax.lax.dot_general(
            x_ref[q], wt_ref[q * _KCH:(q + 1) * _KCH, :],
            (((1,), (0,)), ((), ())),
            preferred_element_type=jnp.float32)
    res = res + b_ref[0][None, :]
    for bi in range(rows // _CPAD):
        o_ref[bi] = lax.slice(
            res, (bi * _CPAD, 0), (bi * _CPAD + cc, res.shape[1])).T


def _head_tc(comb, wt, b2, bb, cc):
    nq, rows_total, kch = comb.shape
    pred = wt.shape[1]
    grid = (bb // _BBLK,)
    rows = _BBLK * _CPAD
    return pl.pallas_call(
        functools.partial(_head_kernel, nq, cc),
        grid=grid,
        in_specs=[
            pl.BlockSpec((nq, rows, kch), lambda t: (0, t, 0)),
            pl.BlockSpec((nq * kch, pred), lambda t: (0, 0)),
            pl.BlockSpec((1, pred), lambda t: (0, 0)),
        ],
        out_specs=pl.BlockSpec((_BBLK, pred, cc), lambda t: (t, 0, 0)),
        out_shape=jax.ShapeDtypeStruct((bb, pred, cc), jnp.float32),
    )(comb, wt, b2)


def kernel(xs, gates, W, b):
    ps, bb, cc, ll, dd = xs.shape
    pred = W.shape[0]
    gbc = jnp.broadcast_to(gates[:, :, None], (bb, ps, _LANES))
    comb = _combine_sc(xs, gbc)
    return _head_tc(comb, W.T, b.reshape(1, pred), bb, cc)


# speedup vs baseline: 4.5407x; 4.5407x over previous
import jax
import jax.numpy as jnp
from jax import lax
from jax.experimental import pallas as pl
from jax.experimental.pallas import tpu as pltpu


def _head_kernel(xs_hbm, g_ref, w_ref, b_ref, o_ref, xbuf, sems):
    c = pl.program_id(0)
    nc = pl.num_programs(0)
    ll = xs_hbm.shape[3]
    ps = xs_hbm.shape[0]

    def copy(slot, cc, i):
        return pltpu.make_async_copy(
            xs_hbm.at[i, :, cc, ll - 1, :], xbuf.at[slot, i], sems.at[slot, i])

    @pl.when(c == 0)
    def _():
        for i in range(ps):
            copy(0, 0, i).start()
        for i in range(ps):
            copy(1, 1, i).start()

    for i in range(ps):
        copy(c % 2, c, i).wait()

    g = jnp.maximum(g_ref[...], 0.0)
    x = xbuf[c % 2]
    comb = x[0] * g[:, 0:1]
    for i in range(1, ps):
        comb = comb + x[i] * g[:, i:i + 1]
    comb = comb + 1e-9

    @pl.when(c + 2 < nc)
    def _():
        for i in range(ps):
            copy(c % 2, c + 2, i).start()

    res = jax.lax.dot_general(
        w_ref[...], comb.T, (((1,), (0,)), ((), ())),
        preferred_element_type=jnp.float32)
    o_ref[0] = res + b_ref[...]


def kernel(xs, gates, W, b):
    ps, bb, cc, ll, dd = xs.shape
    pred = W.shape[0]
    b2 = b.reshape(pred, 1)
    grid = (cc,)
    out_cpb = pl.pallas_call(
        _head_kernel,
        grid=grid,
        in_specs=[
            pl.BlockSpec(memory_space=pl.ANY),
            pl.BlockSpec((bb, ps), lambda t: (0, 0)),
            pl.BlockSpec((pred, dd), lambda t: (0, 0)),
            pl.BlockSpec((pred, 1), lambda t: (0, 0)),
        ],
        out_specs=pl.BlockSpec((1, pred, bb), lambda t: (t, 0, 0)),
        out_shape=jax.ShapeDtypeStruct((cc, pred, bb), jnp.float32),
        scratch_shapes=[
            pltpu.VMEM((2, ps, bb, dd), jnp.float32),
            pltpu.SemaphoreType.DMA((2, ps)),
        ],
    )(xs, gates, W, b2)
    return jnp.transpose(out_cpb, (2, 1, 0))
